# Initial kernel scaffold; baseline (speedup 1.0000x reference)
import jax
import jax.numpy as jnp
from jax.experimental import pallas as pl


def kernel(tile_feat, rr_feat, edge_t2t, edge_rr2t, edge_t2rr, temperature, W1a, b1a, W1b, b1b, W2a, b2a, W2b, b2b, W3a, b3a, W3b, b3b):
    raise NotImplementedError("write your pallas kernel here")


# speedup vs baseline: 1.2483x; 1.2483x over previous
import functools

import jax
import jax.numpy as jnp
from jax.experimental import pallas as pl
from jax.experimental.pallas import tpu as pltpu

D = 128
ACC_W = 144
EDGE_BLK = 512
NODE_BLK = 1000
INTERPRET = False


def _tables_body(xs_ref, xd_ref, wl_ref, wr_ref, b_ref, a_ref, bt_ref):
    a_ref[...] = jnp.dot(xs_ref[...], wl_ref[...],
                         preferred_element_type=jnp.float32)
    bt_ref[...] = jnp.dot(xd_ref[...], wr_ref[...],
                          preferred_element_type=jnp.float32) + b_ref[...]


def _tables(xs, xd, wa, ba):
    n = xs.shape[0]
    wl = wa[:, :D].T
    wr = wa[:, D:].T
    grid = n // NODE_BLK
    return pl.pallas_call(
        _tables_body,
        grid=(grid,),
        in_specs=[
            pl.BlockSpec((NODE_BLK, D), lambda i: (i, 0)),
            pl.BlockSpec((NODE_BLK, D), lambda i: (i, 0)),
            pl.BlockSpec((D, D), lambda i: (0, 0)),
            pl.BlockSpec((D, D), lambda i: (0, 0)),
            pl.BlockSpec((1, D), lambda i: (0, 0)),
        ],
        out_specs=[
            pl.BlockSpec((NODE_BLK, D), lambda i: (i, 0)),
            pl.BlockSpec((NODE_BLK, D), lambda i: (i, 0)),
        ],
        out_shape=[
            jax.ShapeDtypeStruct((n, D), jnp.float32),
            jax.ShapeDtypeStruct((n, D), jnp.float32),
        ],
        interpret=INTERPRET,
    )(xs, xd, wl, wr, ba[None, :])


def _edge_body(n_edges, s_ref, w2_ref, b2_ref, itau_ref, out_ref):
    i = pl.program_id(0)
    h = jnp.maximum(s_ref[...], 0.0)
    m = jnp.dot(h, w2_ref[...], preferred_element_type=jnp.float32) + b2_ref[...]
    nrm = jnp.sqrt(jnp.sum(m * m, axis=1, keepdims=True))
    row = i * EDGE_BLK + jax.lax.broadcasted_iota(jnp.int32, (EDGE_BLK, 1), 0)
    w = jnp.where(row < n_edges, jnp.exp(-nrm * itau_ref[0, 0]), 0.0)
    out_ref[:, :D] = m * w
    lane = jax.lax.broadcasted_iota(jnp.int32, (EDGE_BLK, ACC_W - D), 1)
    out_ref[:, D:] = jnp.where(lane == 0, w, 0.0)


def _edge_mlp(s, wb, bb, inv_tau, n_edges):
    e_pad = s.shape[0]
    grid = e_pad // EDGE_BLK
    return pl.pallas_call(
        functools.partial(_edge_body, n_edges),
        grid=(grid,),
        in_specs=[
            pl.BlockSpec((EDGE_BLK, D), lambda i: (i, 0)),
            pl.BlockSpec((D, D), lambda i: (0, 0)),
            pl.BlockSpec((1, D), lambda i: (0, 0)),
            pl.BlockSpec(memory_space=pltpu.SMEM),
        ],
        out_specs=pl.BlockSpec((EDGE_BLK, ACC_W), lambda i: (i, 0)),
        out_shape=jax.ShapeDtypeStruct((e_pad, ACC_W), jnp.float32),
        interpret=INTERPRET,
    )(s, wb.T, bb[None, :], inv_tau)


def _finalize_body(acc_ref, old_ref, out_ref):
    s = jnp.sum(acc_ref[...], axis=0)
    den = s[:, D:D + 1]
    agg = s[:, :D] / jnp.where(den > 0, den, 1.0)
    out_ref[...] = jnp.where(den > 0, agg, old_ref[...])


def _finalize(acc, old):
    n = old.shape[0]
    p = acc.shape[0]
    grid = n // NODE_BLK
    return pl.pallas_call(
        _finalize_body,
        grid=(grid,),
        in_specs=[
            pl.BlockSpec((p, NODE_BLK, ACC_W), lambda i: (0, i, 0)),
            pl.BlockSpec((NODE_BLK, D), lambda i: (i, 0)),
        ],
        out_specs=pl.BlockSpec((NODE_BLK, D), lambda i: (i, 0)),
        out_shape=jax.ShapeDtypeStruct((n, D), jnp.float32),
        interpret=INTERPRET,
    )(acc, old)


def _pad_idx(idx, e_pad):
    return jnp.concatenate(
        [idx, jnp.zeros((e_pad - idx.shape[0],), dtype=idx.dtype)])


def _stage(xs, xd, src, dst, wa, ba, wb, bb, inv_tau, old, num_dst):
    e = src.shape[0]
    e_pad = ((e + 8191) // 8192) * 8192
    src_p = _pad_idx(src, e_pad)
    dst_p = _pad_idx(dst, e_pad)
    a_tab, b_tab = _tables(xs, xd, wa, ba)
    s = a_tab[src_p] + b_tab[dst_p]
    p_ext = _edge_mlp(s, wb, bb, inv_tau, e)
    acc = jax.ops.segment_sum(p_ext, dst_p, num_segments=num_dst)[None]
    return _finalize(acc, old)


def kernel(tile_feat, rr_feat, edge_t2t, edge_rr2t, edge_t2rr, temperature,
           W1a, b1a, W1b, b1b, W2a, b2a, W2b, b2b, W3a, b3a, W3b, b3b):
    inv_tau = (1.0 / temperature).reshape(1, 1).astype(jnp.float32)
    n_tile = tile_feat.shape[0]
    n_rr = rr_feat.shape[0]
    tile = _stage(tile_feat, tile_feat, edge_t2t[0], edge_t2t[1],
                  W1a, b1a, W1b, b1b, inv_tau, tile_feat, n_tile)
    tile = _stage(rr_feat, tile, edge_rr2t[0], edge_rr2t[1],
                  W2a, b2a, W2b, b2b, inv_tau, tile, n_tile)
    rr = _stage(tile, rr_feat, edge_t2rr[0], edge_t2rr[1],
                W3a, b3a, W3b, b3b, inv_tau, rr_feat, n_rr)
    return tile, rr
